# Initial kernel scaffold; baseline (speedup 1.0000x reference)
import jax
import jax.numpy as jnp
from jax.experimental import pallas as pl


def kernel(input):
    raise NotImplementedError("write your pallas kernel here")


# speedup vs baseline: 23.8140x; 23.8140x over previous
import jax
import jax.numpy as jnp
from jax.experimental import pallas as pl

C_BLK = 8


def _sel_matrix():
    a = jax.lax.broadcasted_iota(jnp.int32, (224, 224), 0)
    b = jax.lax.broadcasted_iota(jnp.int32, (224, 224), 1)
    src = 2 * (b % 112) + b // 112
    return (a == src).astype(jnp.bfloat16)


def _body(x_ref, o_ref):
    v = x_ref[0]
    s = _sel_matrix()
    rs = []
    for u in range(4):
        vu = v[:, :, 224 * u:224 * (u + 1)].reshape(C_BLK * 56, 224)
        hi = vu.astype(jnp.bfloat16)
        lo = (vu - hi.astype(jnp.float32)).astype(jnp.bfloat16)
        r = (jnp.dot(hi, s, preferred_element_type=jnp.float32) +
             jnp.dot(lo, s, preferred_element_type=jnp.float32))
        rs.append(r.reshape(C_BLK, 56, 224))
    outs = []
    for i in range(4):
        kh, kw = i // 2, i % 2
        t0 = rs[2 * kh][:, :, kw * 112:(kw + 1) * 112]
        t1 = rs[2 * kh + 1][:, :, kw * 112:(kw + 1) * 112]
        outs.append(jnp.concatenate([t0, t1], axis=-1))
    o_ref[0] = jnp.stack(outs, axis=0)


def kernel(input):
    n, c, h, w = input.shape
    xv = input.reshape(n, c, h // 4, 4 * w)
    grid = (n, c // C_BLK)
    out = pl.pallas_call(
        _body,
        grid=grid,
        in_specs=[pl.BlockSpec((1, C_BLK, h // 4, 4 * w),
                               lambda ni, ci: (ni, ci, 0, 0))],
        out_specs=pl.BlockSpec((1, 4, C_BLK, h // 4, 2 * (w // 2)),
                               lambda ni, ci: (ni, 0, ci, 0, 0)),
        out_shape=jax.ShapeDtypeStruct((n, 4, c, h // 4, 2 * (w // 2)),
                                       input.dtype),
    )(xv)
    return out.reshape(n, 4 * c, h // 2, w // 2)


# speedup vs baseline: 24.7922x; 1.0411x over previous
import jax
import jax.numpy as jnp
from jax.experimental import pallas as pl

C_BLK = 8


def _row_matrix(kh):
    o = jax.lax.broadcasted_iota(jnp.int32, (112, 224), 0)
    h = jax.lax.broadcasted_iota(jnp.int32, (112, 224), 1)
    src = 4 * (o // 2) + 2 * kh + (o % 2)
    return (h == src).astype(jnp.bfloat16)


def _col_matrix():
    w = jax.lax.broadcasted_iota(jnp.int32, (224, 256), 0)
    b = jax.lax.broadcasted_iota(jnp.int32, (224, 256), 1)
    kw = b // 128
    wo = b % 128
    return ((w == 2 * wo + kw) & (wo < 112)).astype(jnp.bfloat16)


def _split(x):
    hi = x.astype(jnp.bfloat16)
    lo = (x - hi.astype(jnp.float32)).astype(jnp.bfloat16)
    return hi, lo


def _body(x_ref, o_ref):
    kh = pl.program_id(2)
    v = x_ref[0]
    s = _col_matrix()
    for khv in range(2):
        @pl.when(kh == khv)
        def _():
            p = _row_matrix(khv)
            ras = []
            for c in range(C_BLK):
                hi, lo = _split(v[c])
                ra = (jnp.dot(p, hi, preferred_element_type=jnp.float32) +
                      jnp.dot(p, lo, preferred_element_type=jnp.float32))
                ras.append(ra)
            ra_all = jnp.stack(ras, axis=0).reshape(C_BLK * 112, 224)
            hi2, lo2 = _split(ra_all)
            rb = (jnp.dot(hi2, s, preferred_element_type=jnp.float32) +
                  jnp.dot(lo2, s, preferred_element_type=jnp.float32))
            rb = rb.reshape(C_BLK, 112, 256)
            o_ref[0, 0, 0] = rb[:, :, :112]
            o_ref[0, 0, 1] = rb[:, :, 128:240]


def kernel(input):
    n, c, h, w = input.shape
    nc = c // C_BLK
    out = pl.pallas_call(
        _body,
        grid=(n, nc, 2),
        in_specs=[pl.BlockSpec((1, C_BLK, h, w),
                               lambda ni, ci, ki: (ni, ci, 0, 0))],
        out_specs=pl.BlockSpec((1, 1, 2, C_BLK, h // 2, w // 2),
                               lambda ni, ci, ki: (ni, ki, 0, ci, 0, 0)),
        out_shape=jax.ShapeDtypeStruct((n, 2, 2, c, h // 2, w // 2),
                                       input.dtype),
    )(input)
    return out.reshape(n, 4 * c, h // 2, w // 2)
